# baseline (speedup 1.0000x reference)
import functools

import jax
import jax.numpy as jnp
from jax import lax
from jax.experimental import pallas as pl
from jax.experimental.pallas import tpu as pltpu
from jax.experimental.pallas import tpu_sc as plsc

NC = 2
NS = 16
NW = NC * NS
LANES = 16

CW = 256
N_FULL = 3906
TAIL = 1000000 - N_FULL * CW


def _transpose_chunk(bin_, bout, n_cols):
    iota = lax.iota(jnp.int32, LANES)

    @plsc.parallel_loop(0, n_cols // LANES, step=1, unroll=2)
    def _(cb):
        cbase = cb * LANES
        rsplat = jnp.full((LANES,), 0, jnp.int32) + (cb * 8)
        for fb in range(4):
            fbase = fb * LANES
            for k in range(LANES):
                d16 = (iota + k) % LANES
                v = plsc.load_gather(bin_, [fbase + iota, cbase + d16])
                rowv = rsplat + lax.shift_right_logical(d16, 1)
                colv = (d16 & 1) * 64 + fbase + iota
                plsc.store_scatter(bout, [rowv, colv], v)


def _transpose_body(embT_hbm, t2_hbm, bin_a, bin_b, bin_t, bout_a, bout_b,
                    sr_a, sr_b, sw_a, sw_b):
    wid = lax.axis_index("s") * NC + lax.axis_index("c")
    n_w = (N_FULL - 1 - wid) // NW + 1

    def c0_of(j):
        return pl.multiple_of((wid + NW * j) * CW, CW)

    def start_read(j, bin_, sr):
        pltpu.async_copy(embT_hbm.at[:, pl.ds(c0_of(j), CW)], bin_, sr)

    def wait_read(j, bin_, sr):
        pltpu.make_async_copy(embT_hbm.at[:, pl.ds(c0_of(j), CW)], bin_,
                              sr).wait()

    def start_write(j, bout, sw):
        dst0 = pl.multiple_of(c0_of(j) // 2, CW // 2)
        pltpu.async_copy(bout, t2_hbm.at[pl.ds(dst0, CW // 2)], sw)

    def wait_write(bout, sw):
        pltpu.make_async_copy(bout, t2_hbm.at[pl.ds(0, CW // 2)], sw).wait()

    start_read(0, bin_a, sr_a)

    def step(j, carry):
        @pl.when(j % 2 == 0)
        def _():
            @pl.when(j + 1 < n_w)
            def _():
                start_read(j + 1, bin_b, sr_b)
            wait_read(j, bin_a, sr_a)

            @pl.when(j >= 2)
            def _():
                wait_write(bout_a, sw_a)
            _transpose_chunk(bin_a, bout_a, CW)
            start_write(j, bout_a, sw_a)

        @pl.when(j % 2 == 1)
        def _():
            @pl.when(j + 1 < n_w)
            def _():
                start_read(j + 1, bin_a, sr_a)
            wait_read(j, bin_b, sr_b)

            @pl.when(j >= 2)
            def _():
                wait_write(bout_b, sw_b)
            _transpose_chunk(bin_b, bout_b, CW)
            start_write(j, bout_b, sw_b)

        return carry

    lax.fori_loop(0, n_w, step, 0)
    wait_write(bout_a, sw_a)

    @pl.when(n_w >= 2)
    def _():
        wait_write(bout_b, sw_b)

    @pl.when(wid == 0)
    def _():
        c0 = N_FULL * CW
        pltpu.sync_copy(embT_hbm.at[:, pl.ds(c0, TAIL)], bin_t)
        _transpose_chunk(bin_t, bout_a, TAIL)
        pltpu.sync_copy(bout_a.at[pl.ds(0, TAIL // 2)],
                        t2_hbm.at[pl.ds(c0 // 2, TAIL // 2)])


def _gather_body(bpw, L, table2_hbm, wordsT_hbm, out_hbm,
                 idx_v, idxT, gb_a, gb_b, ob_a, ob_b,
                 sg_a, sg_b, sw_a, sw_b):
    wid = lax.axis_index("s") * NC + lax.axis_index("c")
    b0 = pl.multiple_of(wid * bpw, bpw)
    n_pairs = bpw // 2
    n_idx = bpw * L
    NG = (L + LANES - 1) // LANES

    pltpu.sync_copy(wordsT_hbm.at[:, pl.ds(b0, bpw)], idx_v)

    iota = lax.iota(jnp.int32, LANES)

    @plsc.parallel_loop(0, n_idx // LANES, step=1, unroll=8)
    def _(g):
        k = g * LANES + iota
        b = k // L
        l = k - b * L
        iv = plsc.load_gather(idx_v, [l, b])
        plsc.store_scatter(idxT, [b, l], lax.shift_right_logical(iv, 1))

    def start_gather(b, gb, sg):
        pltpu.async_copy(table2_hbm.at[idxT.at[b]], gb, sg)

    def wait_gather(b, gb, sg):
        pltpu.make_async_copy(table2_hbm.at[idxT.at[b]], gb, sg).wait()

    def extract(b, gb, ob):
        bb = jnp.full((LANES,), 0, jnp.int32) + b

        @plsc.parallel_loop(0, L, step=1, unroll=8)
        def _(r):
            rr = jnp.full((LANES,), 0, jnp.int32) + r
            hv = plsc.load_gather(idx_v, [rr, bb])
            pred = (hv & 1) > 0
            for c in range(4):
                v0 = gb[r, pl.ds(c * LANES, LANES)]
                v1 = gb[r, pl.ds(64 + c * LANES, LANES)]
                ob[r, pl.ds(c * LANES, LANES)] = jnp.where(pred, v1, v0)

    def start_wb(b, ob, sw):
        pltpu.async_copy(ob, out_hbm.at[b0 + b], sw)

    def wait_wb(ob, sw):
        pltpu.make_async_copy(ob, out_hbm.at[b0], sw).wait()

    start_gather(0, gb_a, sg_a)

    def pair(p, carry):
        e = p * 2
        o = e + 1

        start_gather(o, gb_b, sg_b)
        wait_gather(e, gb_a, sg_a)

        @pl.when(p >= 1)
        def _():
            wait_wb(ob_a, sw_a)
        extract(e, gb_a, ob_a)
        start_wb(e, ob_a, sw_a)

        @pl.when(p + 1 < n_pairs)
        def _():
            start_gather(e + 2, gb_a, sg_a)

        wait_gather(o, gb_b, sg_b)

        @pl.when(p >= 1)
        def _():
            wait_wb(ob_b, sw_b)
        extract(o, gb_b, ob_b)
        start_wb(o, ob_b, sw_b)
        return carry

    lax.fori_loop(0, n_pairs, pair, 0)
    wait_wb(ob_a, sw_a)
    wait_wb(ob_b, sw_b)


@jax.jit
def kernel(words, word_emb):
    B, L = words.shape
    V, D = word_emb.shape
    if words.dtype != jnp.int32:
        words = words.astype(jnp.int32)

    embT = word_emb.T
    wordsT = words.T

    mesh = plsc.VectorSubcoreMesh(core_axis_name="c", subcore_axis_name="s")

    table2 = pl.kernel(
        _transpose_body,
        out_type=jax.ShapeDtypeStruct((V // 2, 2 * D), jnp.float32),
        mesh=mesh,
        compiler_params=pltpu.CompilerParams(needs_layout_passes=False),
        scratch_types=[
            pltpu.VMEM((D, CW), jnp.float32),
            pltpu.VMEM((D, CW), jnp.float32),
            pltpu.VMEM((D, TAIL), jnp.float32),
            pltpu.VMEM((CW // 2, 2 * D), jnp.float32),
            pltpu.VMEM((CW // 2, 2 * D), jnp.float32),
            pltpu.SemaphoreType.DMA,
            pltpu.SemaphoreType.DMA,
            pltpu.SemaphoreType.DMA,
            pltpu.SemaphoreType.DMA,
        ],
    )(embT)

    bpw = B // NW
    body = functools.partial(_gather_body, bpw, L)
    out = pl.kernel(
        body,
        out_type=jax.ShapeDtypeStruct((B, L, D), jnp.float32),
        mesh=mesh,
        compiler_params=pltpu.CompilerParams(needs_layout_passes=False),
        scratch_types=[
            pltpu.VMEM((L, bpw), jnp.int32),
            pltpu.VMEM((bpw, L), jnp.int32),
            pltpu.VMEM((L, 2 * D), jnp.float32),
            pltpu.VMEM((L, 2 * D), jnp.float32),
            pltpu.VMEM((L, D), jnp.float32),
            pltpu.VMEM((L, D), jnp.float32),
            pltpu.SemaphoreType.DMA,
            pltpu.SemaphoreType.DMA,
            pltpu.SemaphoreType.DMA,
            pltpu.SemaphoreType.DMA,
        ],
    )(table2, wordsT)
    return out


# speedup vs baseline: 1.4422x; 1.4422x over previous
import functools

import jax
import jax.numpy as jnp
from jax import lax
from jax.experimental import pallas as pl
from jax.experimental.pallas import tpu as pltpu
from jax.experimental.pallas import tpu_sc as plsc

NC = 2
NS = 16
NW = NC * NS
LANES = 16

CW = 256
N_FULL = 3906
TAIL = 1000000 - N_FULL * CW


def _lane_shuffle(v, perm):
    return lax.gather(
        v, perm[:, None],
        lax.GatherDimensionNumbers(offset_dims=(), collapsed_slice_dims=(0,),
                                   start_index_map=(0,)),
        slice_sizes=(1,),
        mode=lax.GatherScatterMode.PROMISE_IN_BOUNDS)


def _transpose_chunk(bin_, bout, scr, n_cols):
    iota = lax.iota(jnp.int32, LANES)

    @plsc.parallel_loop(0, n_cols // LANES, step=1, unroll=2)
    def _(cb):
        for fb in range(4):
            fbase = fb * LANES
            for f in range(LANES):
                v = bin_[fbase + f, pl.ds(cb * LANES, LANES)]
                rot = _lane_shuffle(v, (iota + f) % LANES)
                scr[fbase + f, pl.ds(cb * LANES, LANES)] = rot
            for ci in range(LANES):
                c = cb * LANES + ci
                w = plsc.load_gather(
                    scr, [fbase + iota, cb * LANES + (ci - iota) % LANES])
                j = lax.shift_right_logical(c, 1)
                bout[j, pl.ds(((ci % 2) * 64) + fbase, LANES)] = w


def _transpose_body(embT_hbm, t2_hbm, bin_a, bin_b, bin_t, bout_a, bout_b, scr,
                    sr_a, sr_b, sw_a, sw_b):
    wid = lax.axis_index("s") * NC + lax.axis_index("c")
    n_w = (N_FULL - 1 - wid) // NW + 1

    def c0_of(j):
        return pl.multiple_of((wid + NW * j) * CW, CW)

    def start_read(j, bin_, sr):
        pltpu.async_copy(embT_hbm.at[:, pl.ds(c0_of(j), CW)], bin_, sr)

    def wait_read(j, bin_, sr):
        pltpu.make_async_copy(embT_hbm.at[:, pl.ds(c0_of(j), CW)], bin_,
                              sr).wait()

    def start_write(j, bout, sw):
        dst0 = pl.multiple_of(c0_of(j) // 2, CW // 2)
        pltpu.async_copy(bout, t2_hbm.at[pl.ds(dst0, CW // 2)], sw)

    def wait_write(bout, sw):
        pltpu.make_async_copy(bout, t2_hbm.at[pl.ds(0, CW // 2)], sw).wait()

    start_read(0, bin_a, sr_a)

    def step(j, carry):
        @pl.when(j % 2 == 0)
        def _():
            @pl.when(j + 1 < n_w)
            def _():
                start_read(j + 1, bin_b, sr_b)
            wait_read(j, bin_a, sr_a)

            @pl.when(j >= 2)
            def _():
                wait_write(bout_a, sw_a)
            _transpose_chunk(bin_a, bout_a, scr, CW)
            start_write(j, bout_a, sw_a)

        @pl.when(j % 2 == 1)
        def _():
            @pl.when(j + 1 < n_w)
            def _():
                start_read(j + 1, bin_a, sr_a)
            wait_read(j, bin_b, sr_b)

            @pl.when(j >= 2)
            def _():
                wait_write(bout_b, sw_b)
            _transpose_chunk(bin_b, bout_b, scr, CW)
            start_write(j, bout_b, sw_b)

        return carry

    lax.fori_loop(0, n_w, step, 0)
    wait_write(bout_a, sw_a)

    @pl.when(n_w >= 2)
    def _():
        wait_write(bout_b, sw_b)

    @pl.when(wid == 0)
    def _():
        c0 = N_FULL * CW
        pltpu.sync_copy(embT_hbm.at[:, pl.ds(c0, TAIL)], bin_t)
        _transpose_chunk(bin_t, bout_a, scr, TAIL)
        pltpu.sync_copy(bout_a.at[pl.ds(0, TAIL // 2)],
                        t2_hbm.at[pl.ds(c0 // 2, TAIL // 2)])


def _gather_body(bpw, L, table2_hbm, wordsT_hbm, out_hbm,
                 idx_v, idxT, gb_a, gb_b, ob_a, ob_b,
                 sg_a, sg_b, sw_a, sw_b):
    wid = lax.axis_index("s") * NC + lax.axis_index("c")
    b0 = pl.multiple_of(wid * bpw, bpw)
    n_pairs = bpw // 2
    n_idx = bpw * L
    NG = (L + LANES - 1) // LANES

    pltpu.sync_copy(wordsT_hbm.at[:, pl.ds(b0, bpw)], idx_v)

    iota = lax.iota(jnp.int32, LANES)

    @plsc.parallel_loop(0, n_idx // LANES, step=1, unroll=8)
    def _(g):
        k = g * LANES + iota
        b = k // L
        l = k - b * L
        iv = plsc.load_gather(idx_v, [l, b])
        plsc.store_scatter(idxT, [b, l], lax.shift_right_logical(iv, 1))

    def start_gather(b, gb, sg):
        pltpu.async_copy(table2_hbm.at[idxT.at[b]], gb, sg)

    def wait_gather(b, gb, sg):
        pltpu.make_async_copy(table2_hbm.at[idxT.at[b]], gb, sg).wait()

    def extract(b, gb, ob):
        bb = jnp.full((LANES,), 0, jnp.int32) + b

        @plsc.parallel_loop(0, L, step=1, unroll=8)
        def _(r):
            rr = jnp.full((LANES,), 0, jnp.int32) + r
            hv = plsc.load_gather(idx_v, [rr, bb])
            pred = (hv & 1) > 0
            for c in range(4):
                v0 = gb[r, pl.ds(c * LANES, LANES)]
                v1 = gb[r, pl.ds(64 + c * LANES, LANES)]
                ob[r, pl.ds(c * LANES, LANES)] = jnp.where(pred, v1, v0)

    def start_wb(b, ob, sw):
        pltpu.async_copy(ob, out_hbm.at[b0 + b], sw)

    def wait_wb(ob, sw):
        pltpu.make_async_copy(ob, out_hbm.at[b0], sw).wait()

    start_gather(0, gb_a, sg_a)

    def pair(p, carry):
        e = p * 2
        o = e + 1

        start_gather(o, gb_b, sg_b)
        wait_gather(e, gb_a, sg_a)

        @pl.when(p >= 1)
        def _():
            wait_wb(ob_a, sw_a)
        extract(e, gb_a, ob_a)
        start_wb(e, ob_a, sw_a)

        @pl.when(p + 1 < n_pairs)
        def _():
            start_gather(e + 2, gb_a, sg_a)

        wait_gather(o, gb_b, sg_b)

        @pl.when(p >= 1)
        def _():
            wait_wb(ob_b, sw_b)
        extract(o, gb_b, ob_b)
        start_wb(o, ob_b, sw_b)
        return carry

    lax.fori_loop(0, n_pairs, pair, 0)
    wait_wb(ob_a, sw_a)
    wait_wb(ob_b, sw_b)


@jax.jit
def kernel(words, word_emb):
    B, L = words.shape
    V, D = word_emb.shape
    if words.dtype != jnp.int32:
        words = words.astype(jnp.int32)

    embT = word_emb.T
    wordsT = words.T

    mesh = plsc.VectorSubcoreMesh(core_axis_name="c", subcore_axis_name="s")

    table2 = pl.kernel(
        _transpose_body,
        out_type=jax.ShapeDtypeStruct((V // 2, 2 * D), jnp.float32),
        mesh=mesh,
        compiler_params=pltpu.CompilerParams(needs_layout_passes=False),
        scratch_types=[
            pltpu.VMEM((D, CW), jnp.float32),
            pltpu.VMEM((D, CW), jnp.float32),
            pltpu.VMEM((D, TAIL), jnp.float32),
            pltpu.VMEM((CW // 2, 2 * D), jnp.float32),
            pltpu.VMEM((CW // 2, 2 * D), jnp.float32),
            pltpu.VMEM((D, CW), jnp.float32),
            pltpu.SemaphoreType.DMA,
            pltpu.SemaphoreType.DMA,
            pltpu.SemaphoreType.DMA,
            pltpu.SemaphoreType.DMA,
        ],
    )(embT)

    bpw = B // NW
    body = functools.partial(_gather_body, bpw, L)
    out = pl.kernel(
        body,
        out_type=jax.ShapeDtypeStruct((B, L, D), jnp.float32),
        mesh=mesh,
        compiler_params=pltpu.CompilerParams(needs_layout_passes=False),
        scratch_types=[
            pltpu.VMEM((L, bpw), jnp.int32),
            pltpu.VMEM((bpw, L), jnp.int32),
            pltpu.VMEM((L, 2 * D), jnp.float32),
            pltpu.VMEM((L, 2 * D), jnp.float32),
            pltpu.VMEM((L, D), jnp.float32),
            pltpu.VMEM((L, D), jnp.float32),
            pltpu.SemaphoreType.DMA,
            pltpu.SemaphoreType.DMA,
            pltpu.SemaphoreType.DMA,
            pltpu.SemaphoreType.DMA,
        ],
    )(table2, wordsT)
    return out
